# baseline (speedup 1.0000x reference)
import functools

import numpy as np
import jax
import jax.numpy as jnp
from jax import lax
from jax.experimental import pallas as pl
from jax.experimental.pallas import tpu as pltpu
from jax.experimental.pallas import tpu_sc as plsc

NUM_EMB = 1_000_000
D = 64
N_SEQ = 4096
S_LEN = 200
B = N_SEQ * S_LEN
NW = 32
ROWS_PER_W = B // NW
CHUNK = 256
CHUNKS_PER_W = ROWS_PER_W // CHUNK
LANES = 16
TPAD = CHUNK + 1


def _pe_table() -> np.ndarray:
    position = np.arange(S_LEN)[:, None].astype(np.float32)
    div_term = np.exp(np.arange(0, D, 2).astype(np.float32) * (-np.log(10000.0) / D))
    pe = np.zeros((S_LEN, D), dtype=np.float32)
    pe[:, 0::2] = np.sin(position * div_term)
    pe[:, 1::2] = np.cos(position * div_term)
    return pe


_PE = _pe_table()

_mesh = plsc.VectorSubcoreMesh(core_axis_name="c", subcore_axis_name="s")


@functools.partial(
    pl.kernel,
    out_type=jax.ShapeDtypeStruct((S_LEN, D, N_SEQ), jnp.float32),
    mesh=_mesh,
    scratch_types=[
        pltpu.VMEM((ROWS_PER_W,), jnp.int32),
        pltpu.VMEM((S_LEN, D), jnp.float32),
        pltpu.VMEM((CHUNK, D), jnp.float32),
        pltpu.VMEM((CHUNK, D), jnp.float32),
        pltpu.VMEM((D, TPAD), jnp.float32),
        pltpu.VMEM((D, TPAD), jnp.float32),
        pltpu.SemaphoreType.DMA,
        pltpu.SemaphoreType.DMA,
        pltpu.SemaphoreType.DMA,
        pltpu.SemaphoreType.DMA,
    ],
    compiler_params=pltpu.CompilerParams(
        use_tc_tiling_on_sc=False, needs_layout_passes=False,
        disable_bounds_checks=True),
)
def _embed_sc(table_hbm, idx_hbm, pe_hbm, out_hbm, idx_v, pe_v,
              buf0, buf1, tb0, tb1, gs0, gs1, ss0, ss1):
    bufs = (buf0, buf1)
    tbs = (tb0, tb1)
    gsems = (gs0, gs1)
    ssems = (ss0, ss1)
    wid = lax.axis_index("s") * 2 + lax.axis_index("c")
    base = wid * ROWS_PER_W
    pltpu.sync_copy(idx_hbm.at[wid], idx_v)
    pltpu.sync_copy(pe_hbm, pe_v)

    iota = lax.iota(jnp.int32, LANES)
    dvec = [iota + j * LANES for j in range(D // LANES)]

    def gather(ci, k):
        off = pl.multiple_of(ci * CHUNK, 8)
        return pltpu.make_async_copy(
            table_hbm.at[idx_v.at[pl.ds(off, CHUNK)]], bufs[k], gsems[k])

    def store(ci, k):
        t0 = base + ci * CHUNK
        s = t0 // N_SEQ
        n0 = pl.multiple_of(lax.rem(t0, N_SEQ), CHUNK)
        return pltpu.make_async_copy(
            tbs[k].at[:, pl.ds(0, CHUNK)],
            out_hbm.at[s, :, pl.ds(n0, CHUNK)], ssems[k])

    def transpose_add(ci, k):
        buf, tb = bufs[k], tbs[k]
        s = (base + ci * CHUNK) // N_SEQ
        pe_regs = [pe_v[s, pl.ds(j * LANES, LANES)] for j in range(D // LANES)]

        @plsc.parallel_loop(0, CHUNK, unroll=4)
        def row_body(r):
            rvec = jnp.broadcast_to(r, (LANES,))
            for j in range(D // LANES):
                v = buf[r, pl.ds(j * LANES, LANES)] + pe_regs[j]
                plsc.store_scatter(tb, [dvec[j], rvec], v)

    gather(0, 0).start()
    gather(1, 1).start()

    def group_body(g, carry):
        for k in range(2):
            ci = 2 * g + k
            gather(ci, k).wait()

            @pl.when(ci >= 2)
            def _():
                store(ci - 2, k).wait()

            transpose_add(ci, k)
            store(ci, k).start()

            @pl.when(ci <= CHUNKS_PER_W - 3)
            def _():
                gather(ci + 2, k).start()
        return carry

    lax.fori_loop(0, CHUNKS_PER_W // 2, group_body, 0)
    store(CHUNKS_PER_W - 2, 0).wait()
    store(CHUNKS_PER_W - 1, 1).wait()


TCH = 400
TCHUNKS = NUM_EMB // TCH
TROUNDS = TCHUNKS // NW
TBPAD = 65


@functools.partial(
    pl.kernel,
    out_type=jax.ShapeDtypeStruct((NUM_EMB, D), jnp.float32),
    mesh=_mesh,
    scratch_types=[
        pltpu.VMEM((D, TCH), jnp.float32),
        pltpu.VMEM((TCH, TBPAD), jnp.float32),
    ],
    compiler_params=pltpu.CompilerParams(
        use_tc_tiling_on_sc=False, needs_layout_passes=False,
        disable_bounds_checks=True),
)
def _transpose_sc(tt_hbm, out_hbm, bufa, bufb):
    wid = lax.axis_index("s") * 2 + lax.axis_index("c")
    iota = lax.iota(jnp.int32, LANES)
    rvecs = [iota + rb * LANES for rb in range(TCH // LANES)]

    def do_chunk(c):
        r0 = pl.multiple_of(c * TCH, 8)
        pltpu.sync_copy(tt_hbm.at[:, pl.ds(r0, TCH)], bufa)

        @plsc.parallel_loop(0, D, unroll=2)
        def col_body(d):
            dvec2 = jnp.broadcast_to(d, (LANES,))
            for rb in range(TCH // LANES):
                v = bufa[d, pl.ds(rb * LANES, LANES)]
                plsc.store_scatter(bufb, [rvecs[rb], dvec2], v)

        pltpu.sync_copy(bufb.at[:, pl.ds(0, D)], out_hbm.at[pl.ds(r0, TCH)])

    def round_body(k, carry):
        do_chunk(wid + k * NW)
        return carry

    lax.fori_loop(0, TROUNDS, round_body, 0)

    @pl.when(wid < TCHUNKS - TROUNDS * NW)
    def _():
        do_chunk(TROUNDS * NW + wid)


def kernel(pre_embedding, preembed_mask, embed_table):
    idx_t = pre_embedding.astype(jnp.int32).T.reshape(NW, ROWS_PER_W)
    pe = jnp.asarray(_PE)
    table_rm = _transpose_sc(embed_table.T)
    out_t = _embed_sc(table_rm, idx_t, pe)
    return jnp.transpose(out_t, (2, 0, 1)), preembed_mask
